# baseline (speedup 1.0000x reference)
import jax
import jax.numpy as jnp
from jax import lax
from jax.experimental import pallas as pl
from jax.experimental.pallas import tpu as pltpu, tpu_sc as plsc

TOPP = 0.9
V = 100000
VP = 100352
CH = 2048
NCH = VP // CH
NVEC = CH // 16
NW = 32
TB = 65536
MSB = jnp.int32(-2147483648)
NEG_INF = jnp.float32(-jnp.inf)
MIX = jnp.int32(0x5851F42D)


def _mono(xv):
    b = plsc.bitcast(xv, jnp.int32)
    return jnp.where(b < 0, ~b, b ^ MSB)


def _invmono(k):
    b = jnp.where(k < 0, k ^ MSB, ~k)
    return plsc.bitcast(b, jnp.float32)


def _body(x_hbm, g_hbm, out_hbm, y_hbm, k1_hbm, i1_hbm, k2_hbm, i2_hbm,
          tbl, bX, bG, bKi, bIi, wK, wI, wD, oS, semA, semB):
    wid = lax.axis_index("s") * 2 + lax.axis_index("c")
    iota = lax.iota(jnp.int32, 16)
    zero16i = jnp.zeros((16,), jnp.int32)

    def zero_tbl():
        def z(i, _):
            tbl[pl.ds(i * 16, 16)] = zero16i
            return 0
        lax.fori_loop(0, TB // 16, z, 0)

    def scan_tbl():
        def sc(i, carry):
            v = tbl[pl.ds(i * 16, 16)]
            s = plsc.cumsum(v)
            tbl[pl.ds(i * 16, 16)] = s - v + carry
            return carry + jnp.max(s)
        lax.fori_loop(0, TB // 16, sc, jnp.int32(0))

    def cs_k(kv, pos):
        return kv * (2 * pos + 1)

    def cs_i(iv, pos):
        return (iv ^ MIX) * (2 * pos + 97)

    def spin_verify(arrK, arrI, base, expK, expI):
        def one_pass():
            def vch(c, acc):
                pltpu.sync_copy(arrK.at[pl.ds(base + c * CH, CH)], bKi)
                pltpu.sync_copy(arrI.at[pl.ds(base + c * CH, CH)], bIi)

                def vin(j, a2):
                    aK, aI = a2
                    kv = bKi[pl.ds(j * 16, 16)]
                    iv = bIi[pl.ds(j * 16, 16)]
                    pos = iota + (c * CH + j * 16)
                    return (aK + cs_k(kv, pos), aI + cs_i(iv, pos))

                return lax.fori_loop(0, NVEC, vin, acc)

            aK, aI = lax.fori_loop(0, NCH, vch, (zero16i, zero16i))
            return jnp.sum(aK), jnp.sum(aI)

        def wbody(_):
            gK, gI = one_pass()
            bad = jnp.logical_or(gK != expK, gI != expI)
            return jnp.where(bad, jnp.int32(1), jnp.int32(0))

        lax.while_loop(lambda c: c != 0, wbody, jnp.int32(1))

    def row_body(rr, _):
        row = wid * 2 + rr
        xb = row * VP
        ab = wid * VP

        zero_tbl()

        def pA(c, M16):
            pltpu.sync_copy(x_hbm.at[pl.ds(xb + c * CH, CH)], bX)
            pltpu.sync_copy(g_hbm.at[pl.ds(xb + c * CH, CH)], bG)

            def inner(j, Mc):
                xv = bX[pl.ds(j * 16, 16)]
                gv = bG[pl.ds(j * 16, 16)]
                bG[pl.ds(j * 16, 16)] = xv + gv
                key = _mono(xv)
                d1 = jnp.int32(0xFFFF) ^ (key & jnp.int32(0xFFFF))
                cc, ll = plsc.scan_count(d1)
                plsc.addupdate_scatter(tbl, [d1], cc, mask=ll)
                return jnp.maximum(Mc, xv)

            M16 = lax.fori_loop(0, NVEC, inner, M16)
            pltpu.sync_copy(bG, y_hbm.at[pl.ds(ab + c * CH, CH)])
            return M16

        M16 = lax.fori_loop(0, NCH, pA, jnp.full((16,), NEG_INF, jnp.float32))
        M = jnp.max(M16)

        scan_tbl()

        def pB(c, acc):
            pltpu.sync_copy(x_hbm.at[pl.ds(xb + c * CH, CH)], bX)

            def inner(j, a2):
                aK, aI = a2
                xv = bX[pl.ds(j * 16, 16)]
                key = _mono(xv)
                d1 = jnp.int32(0xFFFF) ^ (key & jnp.int32(0xFFFF))
                cc, ll = plsc.scan_count(d1)
                base = plsc.load_gather(tbl, [d1])
                dst = base + cc - 1
                idxv = iota + (c * CH + j * 16)
                wD[pl.ds(j * 16, 16)] = dst + ab
                wK[pl.ds(j * 16, 16)] = key
                wI[pl.ds(j * 16, 16)] = idxv
                plsc.addupdate_scatter(tbl, [d1], cc, mask=ll)
                return (aK + cs_k(key, dst), aI + cs_i(idxv, dst))

            acc = lax.fori_loop(0, NVEC, inner, acc)
            cp1 = pltpu.async_copy(wK, k1_hbm.at[wD], semA)
            cp2 = pltpu.async_copy(wI, i1_hbm.at[wD], semB)
            cp1.wait()
            cp2.wait()
            return acc

        eK16, eI16 = lax.fori_loop(0, NCH, pB, (zero16i, zero16i))
        spin_verify(k1_hbm, i1_hbm, ab, jnp.sum(eK16), jnp.sum(eI16))

        zero_tbl()

        def pC(c, Z16):
            pltpu.sync_copy(k1_hbm.at[pl.ds(ab + c * CH, CH)], bKi)

            def inner(j, Zc):
                key = bKi[pl.ds(j * 16, 16)]
                d2 = jnp.int32(0xFFFF) ^ ((key >> 16) & jnp.int32(0xFFFF))
                cc, ll = plsc.scan_count(d2)
                plsc.addupdate_scatter(tbl, [d2], cc, mask=ll)
                e = jnp.exp(_invmono(key) - M)
                return Zc + e

            return lax.fori_loop(0, NVEC, inner, Z16)

        Z16 = lax.fori_loop(0, NCH, pC, jnp.zeros((16,), jnp.float32))
        thr = jnp.float32(TOPP) * jnp.sum(Z16)

        scan_tbl()

        def pD(c, acc):
            pltpu.sync_copy(k1_hbm.at[pl.ds(ab + c * CH, CH)], bKi)
            pltpu.sync_copy(i1_hbm.at[pl.ds(ab + c * CH, CH)], bIi)

            def inner(j, a2):
                aK, aI = a2
                key = bKi[pl.ds(j * 16, 16)]
                idxv = bIi[pl.ds(j * 16, 16)]
                d2 = jnp.int32(0xFFFF) ^ ((key >> 16) & jnp.int32(0xFFFF))
                cc, ll = plsc.scan_count(d2)
                base = plsc.load_gather(tbl, [d2])
                dst = base + cc - 1
                wD[pl.ds(j * 16, 16)] = dst + ab
                wK[pl.ds(j * 16, 16)] = key
                wI[pl.ds(j * 16, 16)] = idxv
                plsc.addupdate_scatter(tbl, [d2], cc, mask=ll)
                return (aK + cs_k(key, dst), aI + cs_i(idxv, dst))

            acc = lax.fori_loop(0, NVEC, inner, acc)
            cp1 = pltpu.async_copy(wK, k2_hbm.at[wD], semA)
            cp2 = pltpu.async_copy(wI, i2_hbm.at[wD], semB)
            cp1.wait()
            cp2.wait()
            return acc

        eK16, eI16 = lax.fori_loop(0, NCH, pD, (zero16i, zero16i))
        spin_verify(k2_hbm, i2_hbm, ab, jnp.sum(eK16), jnp.sum(eI16))

        def pE1(c, carry):
            pltpu.sync_copy(k2_hbm.at[pl.ds(ab + c * CH, CH)], bKi)

            def inner(j, car):
                cum, cnt = car
                key = bKi[pl.ds(j * 16, 16)]
                e = jnp.exp(_invmono(key) - M)
                s = plsc.cumsum(e) + cum
                pos = iota + (c * CH + j * 16)
                ok = jnp.logical_and(s <= thr, pos <= V - 2)
                cnt = cnt + jnp.where(ok, jnp.int32(1), jnp.int32(0))
                return (jnp.max(s), cnt)

            return lax.fori_loop(0, NVEC, inner, carry)

        _, cnt16 = lax.fori_loop(0, NCH, pE1, (jnp.float32(0.0), zero16i))
        m = jnp.int32(1) + jnp.sum(cnt16)

        def pE2(c, carry):
            pltpu.sync_copy(i2_hbm.at[pl.ds(ab + c * CH, CH)], bIi)
            pltpu.sync_copy(y_hbm.at[pl.ds(ab + c * CH, CH)], bX)

            def inner(j, car):
                best, bestr = car
                jv = bIi[pl.ds(j * 16, 16)]
                yv = bX[pl.ds(j * 16, 16)]
                pos = iota + (c * CH + j * 16)
                keep = jnp.logical_and(jv < m, pos != V - 1)
                upd = jnp.logical_and(keep, yv > best)
                return (jnp.where(upd, yv, best), jnp.where(upd, pos, bestr))

            return lax.fori_loop(0, NVEC, inner, carry)

        best, bestr = lax.fori_loop(
            0, NCH, pE2, (jnp.full((16,), NEG_INF, jnp.float32), zero16i))
        gm = jnp.max(best)
        cand = jnp.where(best == gm, bestr, jnp.int32(2 ** 30))
        rstar = jnp.min(cand)

        oS[...] = jnp.where(iota == 0, rstar, jnp.int32(0))
        pltpu.sync_copy(oS, out_hbm.at[pl.ds(row * 16, 16)])
        return 0

    lax.fori_loop(0, 2, row_body, 0)


def _run(x_flat, g_flat):
    mesh = plsc.VectorSubcoreMesh(core_axis_name="c", subcore_axis_name="s")
    kern = pl.kernel(
        _body,
        out_type=[
            jax.ShapeDtypeStruct((64 * 16,), jnp.int32),
            jax.ShapeDtypeStruct((NW * VP,), jnp.float32),
            jax.ShapeDtypeStruct((NW * VP,), jnp.int32),
            jax.ShapeDtypeStruct((NW * VP,), jnp.int32),
            jax.ShapeDtypeStruct((NW * VP,), jnp.int32),
            jax.ShapeDtypeStruct((NW * VP,), jnp.int32),
        ],
        mesh=mesh,
        compiler_params=pltpu.CompilerParams(needs_layout_passes=False),
        scratch_types=[
            pltpu.VMEM((TB,), jnp.int32),
            pltpu.VMEM((CH,), jnp.float32),
            pltpu.VMEM((CH,), jnp.float32),
            pltpu.VMEM((CH,), jnp.int32),
            pltpu.VMEM((CH,), jnp.int32),
            pltpu.VMEM((CH,), jnp.int32),
            pltpu.VMEM((CH,), jnp.int32),
            pltpu.VMEM((CH,), jnp.int32),
            pltpu.VMEM((16,), jnp.int32),
            pltpu.SemaphoreType.DMA,
            pltpu.SemaphoreType.DMA,
        ],
    )
    return kern(x_flat, g_flat)[0]


def kernel(decoder_out):
    x = decoder_out[0]
    g = jax.random.gumbel(jax.random.key(1), x.shape, jnp.float32)
    xp = jnp.pad(x, ((0, 0), (0, VP - V)), constant_values=-jnp.inf)
    gp = jnp.pad(g, ((0, 0), (0, VP - V)), constant_values=0.0)
    out = _run(xp.reshape(-1), gp.reshape(-1))
    return out.reshape(64, 16)[:, :1].astype(jnp.int64)


# speedup vs baseline: 1.0006x; 1.0006x over previous
import jax
import jax.numpy as jnp
from jax import lax
from jax.experimental import pallas as pl
from jax.experimental.pallas import tpu as pltpu, tpu_sc as plsc

TOPP = 0.9
V = 100000
VP = 100352
CH = 2048
NCH = VP // CH
NVEC = CH // 16
NW = 32
TB = 65536
MSB = jnp.int32(-2147483648)
NEG_INF = jnp.float32(-jnp.inf)
MIX = jnp.int32(0x5851F42D)


def _mono(xv):
    b = plsc.bitcast(xv, jnp.int32)
    return jnp.where(b < 0, ~b, b ^ MSB)


def _invmono(k):
    b = jnp.where(k < 0, k ^ MSB, ~k)
    return plsc.bitcast(b, jnp.float32)


def _cs_k(kv, pos):
    return kv * (2 * pos + 1)


def _cs_i(iv, pos):
    return (iv ^ MIX) * (2 * pos + 97)


def _body(x_hbm, g_hbm, out_hbm, y_hbm, k1_hbm, i1_hbm, k2_hbm, i2_hbm,
          tbl, bX, bG, bKi, bIi,
          wK0, wI0, wD0, wK1, wI1, wD1, oS,
          semA0, semB0, semA1, semB1):
    wid = lax.axis_index("s") * 2 + lax.axis_index("c")
    iota = lax.iota(jnp.int32, 16)
    zero16i = jnp.zeros((16,), jnp.int32)
    wKs, wIs, wDs = [wK0, wK1], [wI0, wI1], [wD0, wD1]
    semAs, semBs = [semA0, semA1], [semB0, semB1]

    def zero_tbl():
        def z(i, _):
            tbl[pl.ds(i * 16, 16)] = zero16i
            return 0
        lax.fori_loop(0, TB // 16, z, 0)

    def scan_tbl():
        def sc(i, carry):
            v = tbl[pl.ds(i * 16, 16)]
            s = plsc.cumsum(v)
            tbl[pl.ds(i * 16, 16)] = s - v + carry
            return carry + jnp.max(s)
        lax.fori_loop(0, TB // 16, sc, jnp.int32(0))

    def row_body(rr, _):
        row = wid * 2 + rr
        xb = row * VP
        ab = wid * VP

        zero_tbl()

        def pA(c, M16):
            pltpu.sync_copy(x_hbm.at[pl.ds(xb + c * CH, CH)], bX)
            pltpu.sync_copy(g_hbm.at[pl.ds(xb + c * CH, CH)], bG)

            def inner(j, Mc):
                xv = bX[pl.ds(j * 16, 16)]
                gv = bG[pl.ds(j * 16, 16)]
                bG[pl.ds(j * 16, 16)] = xv + gv
                key = _mono(xv)
                d1 = jnp.int32(0xFFFF) ^ (key & jnp.int32(0xFFFF))
                cc, ll = plsc.scan_count(d1)
                plsc.addupdate_scatter(tbl, [d1], cc, mask=ll)
                return jnp.maximum(Mc, xv)

            M16 = lax.fori_loop(0, NVEC, inner, M16)
            pltpu.sync_copy(bG, y_hbm.at[pl.ds(ab + c * CH, CH)])
            return M16

        M16 = lax.fori_loop(0, NCH, pA, jnp.full((16,), NEG_INF, jnp.float32))
        M = jnp.max(M16)

        scan_tbl()

        def scatter_pass(load_kv, dst_k_hbm, dst_i_hbm):
            def chunk(c, q, acc, first):
                @pl.when(jnp.logical_not(first))
                def _():
                    pltpu.make_async_copy(
                        wKs[q], dst_k_hbm.at[wDs[q]], semAs[q]).wait()
                    pltpu.make_async_copy(
                        wIs[q], dst_i_hbm.at[wDs[q]], semBs[q]).wait()

                def inner(j, a2):
                    aK, aI = a2
                    key, idxv, dig = load_kv(c, j)
                    cc, ll = plsc.scan_count(dig)
                    base = plsc.load_gather(tbl, [dig])
                    dst = base + cc - 1
                    wDs[q][pl.ds(j * 16, 16)] = dst + ab
                    wKs[q][pl.ds(j * 16, 16)] = key
                    wIs[q][pl.ds(j * 16, 16)] = idxv
                    plsc.addupdate_scatter(tbl, [dig], cc, mask=ll)
                    return (aK + _cs_k(key, dst), aI + _cs_i(idxv, dst))

                acc = lax.fori_loop(0, NVEC, inner, acc)
                pltpu.async_copy(wKs[q], dst_k_hbm.at[wDs[q]], semAs[q])
                pltpu.async_copy(wIs[q], dst_i_hbm.at[wDs[q]], semBs[q])
                return acc

            def pair(p, acc):
                acc = chunk(2 * p, 0, acc, p == 0)
                acc = chunk(2 * p + 1, 1, acc, p == 0)
                return acc

            acc = lax.fori_loop(0, NCH // 2, pair, (zero16i, zero16i))
            acc = chunk(NCH - 1, 0, acc, jnp.bool_(False))
            pltpu.make_async_copy(wKs[0], dst_k_hbm.at[wDs[0]], semAs[0]).wait()
            pltpu.make_async_copy(wIs[0], dst_i_hbm.at[wDs[0]], semBs[0]).wait()
            pltpu.make_async_copy(wKs[1], dst_k_hbm.at[wDs[1]], semAs[1]).wait()
            pltpu.make_async_copy(wIs[1], dst_i_hbm.at[wDs[1]], semBs[1]).wait()
            return acc

        def loadB(c, j):
            xv = bX[pl.ds(j * 16, 16)]
            key = _mono(xv)
            d1 = jnp.int32(0xFFFF) ^ (key & jnp.int32(0xFFFF))
            idxv = iota + (c * CH + j * 16)
            return key, idxv, d1

        def loadB_with_copy(c, j):
            @pl.when(j == 0)
            def _():
                pltpu.sync_copy(x_hbm.at[pl.ds(xb + c * CH, CH)], bX)
            return loadB(c, j)

        eK16, eI16 = scatter_pass(loadB_with_copy, k1_hbm, i1_hbm)
        expK1, expI1 = jnp.sum(eK16), jnp.sum(eI16)

        def runC(_):
            zero_tbl()

            def pC(c, acc):
                pltpu.sync_copy(k1_hbm.at[pl.ds(ab + c * CH, CH)], bKi)
                pltpu.sync_copy(i1_hbm.at[pl.ds(ab + c * CH, CH)], bIi)

                def inner(j, a2):
                    Zc, aK, aI = a2
                    key = bKi[pl.ds(j * 16, 16)]
                    iv = bIi[pl.ds(j * 16, 16)]
                    pos = iota + (c * CH + j * 16)
                    d2 = jnp.int32(0xFFFF) ^ ((key >> 16) & jnp.int32(0xFFFF))
                    cc, ll = plsc.scan_count(d2)
                    plsc.addupdate_scatter(tbl, [d2], cc, mask=ll)
                    e = jnp.exp(_invmono(key) - M)
                    return (Zc + e, aK + _cs_k(key, pos), aI + _cs_i(iv, pos))

                return lax.fori_loop(0, NVEC, inner, acc)

            Z16, aK16, aI16 = lax.fori_loop(
                0, NCH, pC, (jnp.zeros((16,), jnp.float32), zero16i, zero16i))
            bad = jnp.logical_or(jnp.sum(aK16) != expK1,
                                 jnp.sum(aI16) != expI1)
            return (jnp.where(bad, jnp.int32(1), jnp.int32(0)), jnp.sum(Z16))

        _, Z = lax.while_loop(lambda c: c[0] != 0, runC,
                              (jnp.int32(1), jnp.float32(0.0)))
        thr = jnp.float32(TOPP) * Z

        scan_tbl()

        def loadD(c, j):
            @pl.when(j == 0)
            def _():
                pltpu.sync_copy(k1_hbm.at[pl.ds(ab + c * CH, CH)], bKi)
                pltpu.sync_copy(i1_hbm.at[pl.ds(ab + c * CH, CH)], bIi)
            key = bKi[pl.ds(j * 16, 16)]
            idxv = bIi[pl.ds(j * 16, 16)]
            d2 = jnp.int32(0xFFFF) ^ ((key >> 16) & jnp.int32(0xFFFF))
            return key, idxv, d2

        eK16, eI16 = scatter_pass(loadD, k2_hbm, i2_hbm)
        expK2, expI2 = jnp.sum(eK16), jnp.sum(eI16)

        def runE1(_):
            def pE1(c, carry):
                pltpu.sync_copy(k2_hbm.at[pl.ds(ab + c * CH, CH)], bKi)
                pltpu.sync_copy(i2_hbm.at[pl.ds(ab + c * CH, CH)], bIi)

                def inner(j, car):
                    cum, cnt, aK, aI = car
                    key = bKi[pl.ds(j * 16, 16)]
                    iv = bIi[pl.ds(j * 16, 16)]
                    pos = iota + (c * CH + j * 16)
                    e = jnp.exp(_invmono(key) - M)
                    s = plsc.cumsum(e) + cum
                    ok = jnp.logical_and(s <= thr, pos <= V - 2)
                    cnt = cnt + jnp.where(ok, jnp.int32(1), jnp.int32(0))
                    return (jnp.max(s), cnt,
                            aK + _cs_k(key, pos), aI + _cs_i(iv, pos))

                return lax.fori_loop(0, NVEC, inner, carry)

            cum, cnt16, aK16, aI16 = lax.fori_loop(
                0, NCH, pE1,
                (jnp.float32(0.0), zero16i, zero16i, zero16i))
            bad = jnp.logical_or(jnp.sum(aK16) != expK2,
                                 jnp.sum(aI16) != expI2)
            return (jnp.where(bad, jnp.int32(1), jnp.int32(0)),
                    jnp.int32(1) + jnp.sum(cnt16))

        _, m = lax.while_loop(lambda c: c[0] != 0, runE1,
                              (jnp.int32(1), jnp.int32(0)))

        def pE2(c, carry):
            pltpu.sync_copy(i2_hbm.at[pl.ds(ab + c * CH, CH)], bIi)
            pltpu.sync_copy(y_hbm.at[pl.ds(ab + c * CH, CH)], bX)

            def inner(j, car):
                best, bestr = car
                jv = bIi[pl.ds(j * 16, 16)]
                yv = bX[pl.ds(j * 16, 16)]
                pos = iota + (c * CH + j * 16)
                keep = jnp.logical_and(jv < m, pos != V - 1)
                upd = jnp.logical_and(keep, yv > best)
                return (jnp.where(upd, yv, best), jnp.where(upd, pos, bestr))

            return lax.fori_loop(0, NVEC, inner, carry)

        best, bestr = lax.fori_loop(
            0, NCH, pE2, (jnp.full((16,), NEG_INF, jnp.float32), zero16i))
        gm = jnp.max(best)
        cand = jnp.where(best == gm, bestr, jnp.int32(2 ** 30))
        rstar = jnp.min(cand)

        oS[...] = jnp.where(iota == 0, rstar, jnp.int32(0))
        pltpu.sync_copy(oS, out_hbm.at[pl.ds(row * 16, 16)])
        return 0

    lax.fori_loop(0, 2, row_body, 0)


def _run(x_flat, g_flat):
    mesh = plsc.VectorSubcoreMesh(core_axis_name="c", subcore_axis_name="s")
    kern = pl.kernel(
        _body,
        out_type=[
            jax.ShapeDtypeStruct((64 * 16,), jnp.int32),
            jax.ShapeDtypeStruct((NW * VP,), jnp.float32),
            jax.ShapeDtypeStruct((NW * VP,), jnp.int32),
            jax.ShapeDtypeStruct((NW * VP,), jnp.int32),
            jax.ShapeDtypeStruct((NW * VP,), jnp.int32),
            jax.ShapeDtypeStruct((NW * VP,), jnp.int32),
        ],
        mesh=mesh,
        compiler_params=pltpu.CompilerParams(needs_layout_passes=False),
        scratch_types=[
            pltpu.VMEM((TB,), jnp.int32),
            pltpu.VMEM((CH,), jnp.float32),
            pltpu.VMEM((CH,), jnp.float32),
            pltpu.VMEM((CH,), jnp.int32),
            pltpu.VMEM((CH,), jnp.int32),
            pltpu.VMEM((CH,), jnp.int32),
            pltpu.VMEM((CH,), jnp.int32),
            pltpu.VMEM((CH,), jnp.int32),
            pltpu.VMEM((CH,), jnp.int32),
            pltpu.VMEM((CH,), jnp.int32),
            pltpu.VMEM((CH,), jnp.int32),
            pltpu.VMEM((16,), jnp.int32),
            pltpu.SemaphoreType.DMA,
            pltpu.SemaphoreType.DMA,
            pltpu.SemaphoreType.DMA,
            pltpu.SemaphoreType.DMA,
        ],
    )
    return kern(x_flat, g_flat)[0]


def kernel(decoder_out):
    x = decoder_out[0]
    g = jax.random.gumbel(jax.random.key(1), x.shape, jnp.float32)
    xp = jnp.pad(x, ((0, 0), (0, VP - V)), constant_values=-jnp.inf)
    gp = jnp.pad(g, ((0, 0), (0, VP - V)), constant_values=0.0)
    out = _run(xp.reshape(-1), gp.reshape(-1))
    return out.reshape(64, 16)[:, :1].astype(jnp.int64)
